# Initial kernel scaffold; baseline (speedup 1.0000x reference)
import jax
import jax.numpy as jnp
from jax.experimental import pallas as pl


def kernel(x, W_gat0, att_src0, att_dst0, b_gat0, W_gcn1, b_gcn1, Wc, bc, g1, be1, W1, b1, g_o, b_o, W2, b2):
    raise NotImplementedError("write your pallas kernel here")


# speedup vs baseline: 6.5269x; 6.5269x over previous
import functools

import jax
import jax.numpy as jnp
from jax.experimental import pallas as pl
from jax.experimental.pallas import tpu as pltpu

N_NODES = 52
EPS = 1e-5
C1 = 64
C2 = 128
BT = 256


def _fused_kernel(xt_ref, w01_ref, c0_ref, wc_ref, c1_ref, w1_ref, b1_ref,
                  w2_ref, b2_ref, out_ref):
    bt = xt_ref.shape[0]
    x2 = xt_ref[...].reshape(bt * N_NODES, 14)
    h = jnp.dot(x2, w01_ref[...], preferred_element_type=jnp.float32)
    h = jnp.maximum(h + c0_ref[...], 0.0)
    h3 = h.reshape(bt, N_NODES, C2)
    y = jax.lax.dot_general(h3, wc_ref[...], (((1,), (1,)), ((), ())),
                            preferred_element_type=jnp.float32)
    y = jnp.maximum(y + c1_ref[...][None], 0.0)
    yf = y.reshape(bt, C2 * N_NODES)
    z = jnp.dot(yf, w1_ref[...], preferred_element_type=jnp.float32)
    z = jnp.maximum(z + b1_ref[...], 0.0)
    out_ref[...] = jnp.dot(z, w2_ref[...],
                           preferred_element_type=jnp.float32) + b2_ref[...]


@jax.jit
def kernel(x, W_gat0, att_src0, att_dst0, b_gat0, W_gcn1, b_gcn1, Wc, bc,
           g1, be1, W1, b1, g_o, b_o, W2, b2):
    B = x.shape[0]
    inv = 1.0 / jnp.sqrt(jnp.float32(1.0 + EPS))
    W01 = W_gat0 @ W_gcn1
    c0 = b_gat0 @ W_gcn1 + b_gcn1
    Wc2 = (g1 * inv)[:, None] * Wc
    c1 = bc * g1 * inv + be1
    W1s = W1 * (g_o * inv)[None, :]
    W1p = W1s.reshape(N_NODES, C2, 64).transpose(1, 0, 2).reshape(C2 * N_NODES, 64)
    b1s = b1 * g_o * inv + b_o

    xt = x.transpose(0, 2, 1)
    num_tiles = B // BT

    grid_spec = pltpu.PrefetchScalarGridSpec(
        num_scalar_prefetch=0,
        grid=(num_tiles,),
        in_specs=[
            pl.BlockSpec((BT, N_NODES, 14), lambda i: (i, 0, 0)),
            pl.BlockSpec((14, C2), lambda i: (0, 0)),
            pl.BlockSpec((1, C2), lambda i: (0, 0)),
            pl.BlockSpec((N_NODES, N_NODES), lambda i: (0, 0)),
            pl.BlockSpec((1, N_NODES), lambda i: (0, 0)),
            pl.BlockSpec((C2 * N_NODES, 64), lambda i: (0, 0)),
            pl.BlockSpec((1, 64), lambda i: (0, 0)),
            pl.BlockSpec((64, 10), lambda i: (0, 0)),
            pl.BlockSpec((1, 10), lambda i: (0, 0)),
        ],
        out_specs=pl.BlockSpec((BT, 10), lambda i: (i, 0)),
    )
    out = pl.pallas_call(
        _fused_kernel,
        grid_spec=grid_spec,
        out_shape=jax.ShapeDtypeStruct((B, 10), jnp.float32),
        compiler_params=pltpu.CompilerParams(
            dimension_semantics=("arbitrary",),
        ),
    )(xt, W01, c0.reshape(1, C2), Wc2, c1.reshape(1, N_NODES), W1p,
      b1s.reshape(1, 64), W2, b2.reshape(1, 10))
    return out


# speedup vs baseline: 8.1639x; 1.2508x over previous
import functools

import jax
import jax.numpy as jnp
from jax.experimental import pallas as pl
from jax.experimental.pallas import tpu as pltpu

N_NODES = 52
EPS = 1e-5
C1 = 64
C2 = 128
BT = 256


def _fused_kernel(x_ref, w01_ref, c0_ref, wc_ref, c1_ref, w1_ref, b1_ref,
                  w2_ref, b2_ref, out_ref):
    bt = x_ref.shape[0]
    h3 = jax.lax.dot_general(x_ref[...], w01_ref[...], (((1,), (0,)), ((), ())),
                             preferred_element_type=jnp.float32)
    h3 = jnp.maximum(h3 + c0_ref[...][None], 0.0)
    y = jax.lax.dot_general(wc_ref[...], h3.astype(jnp.bfloat16),
                            (((1,), (1,)), ((), ())),
                            preferred_element_type=jnp.float32)
    y = jnp.maximum(y + c1_ref[...][:, :, None], 0.0)
    yb = y.astype(jnp.bfloat16)
    z = jnp.zeros((bt, 64), jnp.float32)
    for o in range(N_NODES):
        z = z + jnp.dot(yb[o], w1_ref[o], preferred_element_type=jnp.float32)
    z = jnp.maximum(z + b1_ref[...], 0.0)
    out_ref[...] = jnp.dot(z, w2_ref[...],
                           preferred_element_type=jnp.float32) + b2_ref[...]


@jax.jit
def kernel(x, W_gat0, att_src0, att_dst0, b_gat0, W_gcn1, b_gcn1, Wc, bc,
           g1, be1, W1, b1, g_o, b_o, W2, b2):
    B = x.shape[0]
    inv = 1.0 / jnp.sqrt(jnp.float32(1.0 + EPS))
    W01 = W_gat0 @ W_gcn1
    c0 = b_gat0 @ W_gcn1 + b_gcn1
    Wc2 = (g1 * inv)[:, None] * Wc
    c1 = bc * g1 * inv + be1
    W1s = W1 * (g_o * inv)[None, :]
    W1p = W1s.reshape(N_NODES, C2, 64)
    b1s = b1 * g_o * inv + b_o

    num_tiles = B // BT

    grid_spec = pltpu.PrefetchScalarGridSpec(
        num_scalar_prefetch=0,
        grid=(num_tiles,),
        in_specs=[
            pl.BlockSpec((BT, 14, N_NODES), lambda i: (i, 0, 0)),
            pl.BlockSpec((14, C2), lambda i: (0, 0)),
            pl.BlockSpec((1, C2), lambda i: (0, 0)),
            pl.BlockSpec((N_NODES, N_NODES), lambda i: (0, 0)),
            pl.BlockSpec((N_NODES, 1), lambda i: (0, 0)),
            pl.BlockSpec((N_NODES, C2, 64), lambda i: (0, 0, 0)),
            pl.BlockSpec((1, 64), lambda i: (0, 0)),
            pl.BlockSpec((64, 10), lambda i: (0, 0)),
            pl.BlockSpec((1, 10), lambda i: (0, 0)),
        ],
        out_specs=pl.BlockSpec((BT, 10), lambda i: (i, 0)),
    )
    out = pl.pallas_call(
        _fused_kernel,
        grid_spec=grid_spec,
        out_shape=jax.ShapeDtypeStruct((B, 10), jnp.float32),
        compiler_params=pltpu.CompilerParams(
            dimension_semantics=("arbitrary",),
        ),
    )(x, W01, c0.reshape(1, C2),
      Wc2.astype(jnp.bfloat16),
      c1.reshape(N_NODES, 1), W1p.astype(jnp.bfloat16),
      b1s.reshape(1, 64), W2, b2.reshape(1, 10))
    return out
